# Initial kernel scaffold; baseline (speedup 1.0000x reference)
import jax
import jax.numpy as jnp
from jax.experimental import pallas as pl


def kernel(x, edge_index, edge_weight, W, b):
    raise NotImplementedError("write your pallas kernel here")


# speedup vs baseline: 2.6094x; 2.6094x over previous
import functools

import jax
import jax.numpy as jnp
from jax import lax
from jax.experimental import pallas as pl
from jax.experimental.pallas import tpu as pltpu
from jax.experimental.pallas import tpu_sc as plsc

NC = 2
NS = 16
L = 16
NW = NC * NS


def _rsqrt_newton(d):
    bits = lax.bitcast_convert_type(d, jnp.int32)
    y = lax.bitcast_convert_type(
        jnp.int32(0x5F3759DF) - lax.shift_right_logical(bits, 1), jnp.float32)
    for _ in range(3):
        y = y * (1.5 - 0.5 * d * y * y)
    return y


def _zero_1d(ref, n):
    def z(i, _):
        ref[pl.ds(i * L, L)] = jnp.zeros((L,), jnp.float32)
        return 0
    lax.fori_loop(0, n // L, z, 0)


def _make_norm_kernel(E_pad, NP):
    EPT = E_pad // NS
    EPW = E_pad // NW
    SL = NP // NS
    mesh = plsc.VectorSubcoreMesh(
        core_axis_name="c", subcore_axis_name="s",
        num_cores=NC, num_subcores=NS)

    @functools.partial(
        pl.kernel, mesh=mesh,
        compiler_params=pltpu.CompilerParams(needs_layout_passes=False),
        out_type=jax.ShapeDtypeStruct((E_pad,), jnp.float32),
        scratch_types=[
            pltpu.VMEM((NP,), jnp.float32),
            pltpu.VMEM((NP,), jnp.float32),
            pltpu.VMEM((EPT,), jnp.int32),
            pltpu.VMEM((EPT,), jnp.float32),
            pltpu.VMEM((EPW,), jnp.int32),
            pltpu.VMEM((EPW,), jnp.float32),
            pltpu.VMEM((SL,), jnp.float32),
            pltpu.VMEM((SL,), jnp.float32),
            pltpu.VMEM_SHARED((NS, NP), jnp.float32),
            pltpu.VMEM_SHARED((NP,), jnp.float32),
        ],
    )
    def norm_kernel(row_hbm, col_hbm, w_hbm, norm_hbm,
                    deg_l, dinv_l, row_b, w_b, col_b, norm_b, tmp_b, acc_b,
                    deg_sh, dinv_sh):
        c = lax.axis_index("c")
        s = lax.axis_index("s")
        wid = s * NC + c

        _zero_1d(deg_l, NP)
        pltpu.sync_copy(row_hbm.at[pl.ds(s * EPT, EPT)], row_b)
        pltpu.sync_copy(w_hbm.at[pl.ds(s * EPT, EPT)], w_b)

        def acc_deg(g, _):
            r = row_b[pl.ds(g * L, L)]
            w = w_b[pl.ds(g * L, L)]
            plsc.addupdate_scatter(deg_l, [r], w)
            return 0
        lax.fori_loop(0, EPT // L, acc_deg, 0)
        pltpu.sync_copy(deg_l, deg_sh.at[s])
        plsc.subcore_barrier()

        base = s * SL
        _zero_1d(acc_b, SL)

        def red(j, _):
            pltpu.sync_copy(deg_sh.at[j, pl.ds(base, SL)], tmp_b)

            def addg(g, _):
                sl = pl.ds(g * L, L)
                acc_b[sl] = acc_b[sl] + tmp_b[sl]
                return 0
            lax.fori_loop(0, SL // L, addg, 0)
            return 0
        lax.fori_loop(0, NS, red, 0)

        def din(g, _):
            sl = pl.ds(g * L, L)
            d = acc_b[sl]
            acc_b[sl] = jnp.where(d > 0.0, _rsqrt_newton(d), 0.0)
            return 0
        lax.fori_loop(0, SL // L, din, 0)
        pltpu.sync_copy(acc_b, dinv_sh.at[pl.ds(base, SL)])
        plsc.subcore_barrier()

        pltpu.sync_copy(dinv_sh, dinv_l)
        ebase = wid * EPW
        pltpu.sync_copy(row_hbm.at[pl.ds(ebase, EPW)], row_b.at[pl.ds(0, EPW)])
        pltpu.sync_copy(col_hbm.at[pl.ds(ebase, EPW)], col_b)
        pltpu.sync_copy(w_hbm.at[pl.ds(ebase, EPW)], w_b.at[pl.ds(0, EPW)])

        def nrm(g, _):
            sl = pl.ds(g * L, L)
            dr = plsc.load_gather(dinv_l, [row_b[sl]])
            dc = plsc.load_gather(dinv_l, [col_b[sl]])
            norm_b[sl] = (-w_b[sl]) * dr * dc
            return 0
        lax.fori_loop(0, EPW // L, nrm, 0)
        pltpu.sync_copy(norm_b, norm_hbm.at[pl.ds(ebase, EPW)])

    return norm_kernel


def _make_prop_kernel(E_pad, NP, F, C):
    FPW = F // NW
    FG = 4
    assert FPW % FG == 0
    mesh = plsc.VectorSubcoreMesh(
        core_axis_name="c", subcore_axis_name="s",
        num_cores=NC, num_subcores=NS)

    @functools.partial(
        pl.kernel, mesh=mesh,
        compiler_params=pltpu.CompilerParams(needs_layout_passes=False),
        out_type=(jax.ShapeDtypeStruct((F, NP), jnp.float32),
                  jax.ShapeDtypeStruct((F, NP), jnp.float32)),
        scratch_types=(
            [pltpu.VMEM((NP,), jnp.float32) for _ in range(FG)] +
            [pltpu.VMEM((NP,), jnp.float32) for _ in range(FG)] +
            [pltpu.VMEM((NP,), jnp.float32),
             pltpu.VMEM((C,), jnp.int32),
             pltpu.VMEM((C,), jnp.int32),
             pltpu.VMEM((C,), jnp.float32)]
        ),
    )
    def prop_kernel(xT, row_hbm, col_hbm, norm_hbm, t1T, t2T,
                    a0, a1, a2, a3, b0, b1, b2, b3, xtmp, rb, cb, nb):
        A = [a0, a1, a2, a3]
        B = [b0, b1, b2, b3]
        c = lax.axis_index("c")
        s = lax.axis_index("s")
        wid = s * NC + c
        f0 = wid * FPW

        def edge_sweep(src, dst):
            def chunk(ci, _):
                eb = ci * C
                pltpu.sync_copy(row_hbm.at[pl.ds(eb, C)], rb)
                pltpu.sync_copy(col_hbm.at[pl.ds(eb, C)], cb)
                pltpu.sync_copy(norm_hbm.at[pl.ds(eb, C)], nb)

                def grp(g, _):
                    sl = pl.ds(g * L, L)
                    r = rb[sl]
                    cc = cb[sl]
                    nv = nb[sl]
                    for ff in range(FG):
                        v = plsc.load_gather(src[ff], [cc])
                        plsc.addupdate_scatter(dst[ff], [r], v * nv)
                    return 0
                lax.fori_loop(0, C // L, grp, 0)
                return 0
            lax.fori_loop(0, E_pad // C, chunk, 0)

        for fg in range(FPW // FG):
            fbase = f0 + fg * FG
            for ff in range(FG):
                pltpu.sync_copy(xT.at[fbase + ff], A[ff])
                _zero_1d(B[ff], NP)
            edge_sweep(A, B)
            for ff in range(FG):
                pltpu.sync_copy(B[ff], t1T.at[fbase + ff])
            for ff in range(FG):
                _zero_1d(A[ff], NP)
            edge_sweep(B, A)
            for ff in range(FG):
                pltpu.sync_copy(xT.at[fbase + ff], xtmp)
                a = A[ff]

                def comb(g, _):
                    sl = pl.ds(g * L, L)
                    a[sl] = 2.0 * a[sl] - xtmp[sl]
                    return 0
                lax.fori_loop(0, NP // L, comb, 0)
                pltpu.sync_copy(a, t2T.at[fbase + ff])

    return prop_kernel


def _matmul_relu(xT, t1T, t2T, W, b, N, NP, F_IN, F_OUT, BN=1024):
    def mm_kernel(x_ref, t1_ref, t2_ref, w0, w1, w2, b_ref, o_ref):
        dn = (((0,), (0,)), ((), ()))
        acc = lax.dot_general(x_ref[...], w0[...], dn,
                              preferred_element_type=jnp.float32)
        acc = acc + lax.dot_general(t1_ref[...], w1[...], dn,
                                    preferred_element_type=jnp.float32)
        acc = acc + lax.dot_general(t2_ref[...], w2[...], dn,
                                    preferred_element_type=jnp.float32)
        acc = acc + b_ref[...]
        o_ref[...] = jnp.maximum(acc, 0.0)

    grid = (pl.cdiv(N, BN),)
    fspec = pl.BlockSpec((F_IN, BN), lambda i: (0, i))
    wspec = pl.BlockSpec((F_IN, F_OUT), lambda i: (0, 0))
    return pl.pallas_call(
        mm_kernel,
        grid=grid,
        in_specs=[fspec, fspec, fspec, wspec, wspec, wspec,
                  pl.BlockSpec((1, F_OUT), lambda i: (0, 0))],
        out_specs=pl.BlockSpec((BN, F_OUT), lambda i: (i, 0)),
        out_shape=jax.ShapeDtypeStruct((N, F_OUT), jnp.float32),
    )(xT, t1T, t2T, W[0], W[1], W[2], b.reshape(1, F_OUT))


def kernel(x, edge_index, edge_weight, W, b):
    N, F_IN = x.shape
    F_OUT = W.shape[2]
    E = edge_weight.shape[0]

    E_pad = -(-E // (NW * 128)) * (NW * 128)
    pad = E_pad - E
    row_p = jnp.pad(edge_index[0], (0, pad))
    col_p = jnp.pad(edge_index[1], (0, pad))
    w_p = jnp.pad(edge_weight, (0, pad))
    NP = -(-N // (NS * L)) * (NS * L)

    norm = _make_norm_kernel(E_pad, NP)(row_p, col_p, w_p)
    xTp = jnp.pad(x.T, ((0, 0), (0, NP - N)))
    t1T, t2T = _make_prop_kernel(E_pad, NP, F_IN, C=E_pad // 32)(
        xTp, row_p, col_p, norm)
    return _matmul_relu(xTp, t1T, t2T, W, b, N, NP, F_IN, F_OUT)


# speedup vs baseline: 3.2531x; 1.2467x over previous
import functools

import jax
import jax.numpy as jnp
from jax import lax
from jax.experimental import pallas as pl
from jax.experimental.pallas import tpu as pltpu
from jax.experimental.pallas import tpu_sc as plsc

NC = 2
NS = 16
L = 16
NW = NC * NS


def _rsqrt_newton(d):
    bits = lax.bitcast_convert_type(d, jnp.int32)
    y = lax.bitcast_convert_type(
        jnp.int32(0x5F3759DF) - lax.shift_right_logical(bits, 1), jnp.float32)
    for _ in range(3):
        y = y * (1.5 - 0.5 * d * y * y)
    return y


def _zero_1d(ref, n):
    def z(i, _):
        ref[pl.ds(i * L, L)] = jnp.zeros((L,), jnp.float32)
        return 0
    lax.fori_loop(0, n // L, z, 0, unroll=8)


def _make_norm_kernel(E_pad, NP):
    EPT = E_pad // NS
    EPW = E_pad // NW
    SL = NP // NS
    mesh = plsc.VectorSubcoreMesh(
        core_axis_name="c", subcore_axis_name="s",
        num_cores=NC, num_subcores=NS)

    @functools.partial(
        pl.kernel, mesh=mesh,
        compiler_params=pltpu.CompilerParams(needs_layout_passes=False),
        out_type=jax.ShapeDtypeStruct((E_pad,), jnp.float32),
        scratch_types=[
            pltpu.VMEM((NP,), jnp.float32),
            pltpu.VMEM((NP,), jnp.float32),
            pltpu.VMEM((EPT,), jnp.int32),
            pltpu.VMEM((EPT,), jnp.float32),
            pltpu.VMEM((EPW,), jnp.int32),
            pltpu.VMEM((EPW,), jnp.float32),
            pltpu.VMEM((SL,), jnp.float32),
            pltpu.VMEM((SL,), jnp.float32),
            pltpu.VMEM_SHARED((NS, NP), jnp.float32),
            pltpu.VMEM_SHARED((NP,), jnp.float32),
        ],
    )
    def norm_kernel(row_hbm, col_hbm, w_hbm, norm_hbm,
                    deg_l, dinv_l, row_b, w_b, col_b, norm_b, tmp_b, acc_b,
                    deg_sh, dinv_sh):
        c = lax.axis_index("c")
        s = lax.axis_index("s")
        wid = s * NC + c

        _zero_1d(deg_l, NP)
        pltpu.sync_copy(row_hbm.at[pl.ds(s * EPT, EPT)], row_b)
        pltpu.sync_copy(w_hbm.at[pl.ds(s * EPT, EPT)], w_b)

        def acc_deg(g, _):
            r = row_b[pl.ds(g * L, L)]
            w = w_b[pl.ds(g * L, L)]
            plsc.addupdate_scatter(deg_l, [r], w)
            return 0
        lax.fori_loop(0, EPT // L, acc_deg, 0)
        pltpu.sync_copy(deg_l, deg_sh.at[s])
        plsc.subcore_barrier()

        base = s * SL
        _zero_1d(acc_b, SL)

        def red(j, _):
            pltpu.sync_copy(deg_sh.at[j, pl.ds(base, SL)], tmp_b)

            def addg(g, _):
                sl = pl.ds(g * L, L)
                acc_b[sl] = acc_b[sl] + tmp_b[sl]
                return 0
            lax.fori_loop(0, SL // L, addg, 0)
            return 0
        lax.fori_loop(0, NS, red, 0)

        def din(g, _):
            sl = pl.ds(g * L, L)
            d = acc_b[sl]
            acc_b[sl] = jnp.where(d > 0.0, _rsqrt_newton(d), 0.0)
            return 0
        lax.fori_loop(0, SL // L, din, 0)
        pltpu.sync_copy(acc_b, dinv_sh.at[pl.ds(base, SL)])
        plsc.subcore_barrier()

        pltpu.sync_copy(dinv_sh, dinv_l)
        ebase = wid * EPW
        pltpu.sync_copy(row_hbm.at[pl.ds(ebase, EPW)], row_b.at[pl.ds(0, EPW)])
        pltpu.sync_copy(col_hbm.at[pl.ds(ebase, EPW)], col_b)
        pltpu.sync_copy(w_hbm.at[pl.ds(ebase, EPW)], w_b.at[pl.ds(0, EPW)])

        def nrm(g, _):
            sl = pl.ds(g * L, L)
            dr = plsc.load_gather(dinv_l, [row_b[sl]])
            dc = plsc.load_gather(dinv_l, [col_b[sl]])
            norm_b[sl] = (-w_b[sl]) * dr * dc
            return 0
        lax.fori_loop(0, EPW // L, nrm, 0)
        pltpu.sync_copy(norm_b, norm_hbm.at[pl.ds(ebase, EPW)])

    return norm_kernel


def _make_prop_kernel(E_pad, NP, F, C):
    FPW = F // NW
    FG = 4
    assert FPW % FG == 0
    NCH = E_pad // C
    assert NCH % 2 == 0
    mesh = plsc.VectorSubcoreMesh(
        core_axis_name="c", subcore_axis_name="s",
        num_cores=NC, num_subcores=NS)

    @functools.partial(
        pl.kernel, mesh=mesh,
        compiler_params=pltpu.CompilerParams(needs_layout_passes=False),
        out_type=(jax.ShapeDtypeStruct((F, NP), jnp.float32),
                  jax.ShapeDtypeStruct((F, NP), jnp.float32)),
        scratch_types=(
            [pltpu.VMEM((NP,), jnp.float32) for _ in range(2 * FG)] +
            [pltpu.VMEM((C,), jnp.int32) for _ in range(2)] +
            [pltpu.VMEM((C,), jnp.int32) for _ in range(2)] +
            [pltpu.VMEM((C,), jnp.float32) for _ in range(2)] +
            [pltpu.SemaphoreType.DMA for _ in range(6)]
        ),
    )
    def prop_kernel(xT, row_hbm, col_hbm, norm_hbm, t1T, t2T,
                    a0, a1, a2, a3, b0, b1, b2, b3,
                    rb0, rb1, cb0, cb1, nb0, nb1,
                    sr0, sr1, sc0, sc1, sn0, sn1):
        A = [a0, a1, a2, a3]
        B = [b0, b1, b2, b3]
        rbs, cbs, nbs = [rb0, rb1], [cb0, cb1], [nb0, nb1]
        srs, scs, sns = [sr0, sr1], [sc0, sc1], [sn0, sn1]
        c = lax.axis_index("c")
        s = lax.axis_index("s")
        wid = s * NC + c
        f0 = wid * FPW

        def start(ci, slot):
            eb = ci * C
            pltpu.async_copy(row_hbm.at[pl.ds(eb, C)], rbs[slot], srs[slot])
            pltpu.async_copy(col_hbm.at[pl.ds(eb, C)], cbs[slot], scs[slot])
            pltpu.async_copy(norm_hbm.at[pl.ds(eb, C)], nbs[slot], sns[slot])

        def wait(slot):
            pltpu.make_async_copy(
                row_hbm.at[pl.ds(0, C)], rbs[slot], srs[slot]).wait()
            pltpu.make_async_copy(
                col_hbm.at[pl.ds(0, C)], cbs[slot], scs[slot]).wait()
            pltpu.make_async_copy(
                norm_hbm.at[pl.ds(0, C)], nbs[slot], sns[slot]).wait()

        def edge_sweep(src, dst):
            def process(slot):
                rb, cb, nb = rbs[slot], cbs[slot], nbs[slot]

                def grp(g, _):
                    sl = pl.ds(g * L, L)
                    r = rb[sl]
                    cc = cb[sl]
                    nv = nb[sl]
                    for ff in range(FG):
                        v = plsc.load_gather(src[ff], [cc])
                        plsc.addupdate_scatter(dst[ff], [r], v * nv)
                    return 0
                lax.fori_loop(0, C // L, grp, 0, unroll=8)

            start(0, 0)

            def body2(ci2, _):
                ci = ci2 * 2
                start(ci + 1, 1)
                wait(0)
                process(0)

                @pl.when(ci2 < NCH // 2 - 1)
                def _():
                    start(ci + 2, 0)
                wait(1)
                process(1)
                return 0
            lax.fori_loop(0, NCH // 2, body2, 0)

        for fg in range(FPW // FG):
            fbase = f0 + fg * FG
            for ff in range(FG):
                pltpu.sync_copy(xT.at[fbase + ff], A[ff])
                _zero_1d(B[ff], NP)
            edge_sweep(A, B)
            for ff in range(FG):
                pltpu.sync_copy(B[ff], t1T.at[fbase + ff])
            for ff in range(FG):
                _zero_1d(A[ff], NP)
            edge_sweep(B, A)
            for ff in range(FG):
                pltpu.sync_copy(xT.at[fbase + ff], B[ff])
                a, xt = A[ff], B[ff]

                def comb(g, _):
                    sl = pl.ds(g * L, L)
                    a[sl] = 2.0 * a[sl] - xt[sl]
                    return 0
                lax.fori_loop(0, NP // L, comb, 0, unroll=8)
                pltpu.sync_copy(a, t2T.at[fbase + ff])

    return prop_kernel


def _matmul_relu(xT, t1T, t2T, W, b, N, NP, F_IN, F_OUT, BN=1024):
    def mm_kernel(x_ref, t1_ref, t2_ref, w0, w1, w2, b_ref, o_ref):
        dn = (((0,), (0,)), ((), ()))
        acc = lax.dot_general(x_ref[...], w0[...], dn,
                              preferred_element_type=jnp.float32)
        acc = acc + lax.dot_general(t1_ref[...], w1[...], dn,
                                    preferred_element_type=jnp.float32)
        acc = acc + lax.dot_general(t2_ref[...], w2[...], dn,
                                    preferred_element_type=jnp.float32)
        acc = acc + b_ref[...]
        o_ref[...] = jnp.maximum(acc, 0.0)

    grid = (pl.cdiv(N, BN),)
    fspec = pl.BlockSpec((F_IN, BN), lambda i: (0, i))
    wspec = pl.BlockSpec((F_IN, F_OUT), lambda i: (0, 0))
    return pl.pallas_call(
        mm_kernel,
        grid=grid,
        in_specs=[fspec, fspec, fspec, wspec, wspec, wspec,
                  pl.BlockSpec((1, F_OUT), lambda i: (0, 0))],
        out_specs=pl.BlockSpec((BN, F_OUT), lambda i: (i, 0)),
        out_shape=jax.ShapeDtypeStruct((N, F_OUT), jnp.float32),
    )(xT, t1T, t2T, W[0], W[1], W[2], b.reshape(1, F_OUT))


def kernel(x, edge_index, edge_weight, W, b):
    N, F_IN = x.shape
    F_OUT = W.shape[2]
    E = edge_weight.shape[0]

    E_pad = -(-E // (NW * 128)) * (NW * 128)
    pad = E_pad - E
    row_p = jnp.pad(edge_index[0], (0, pad))
    col_p = jnp.pad(edge_index[1], (0, pad))
    w_p = jnp.pad(edge_weight, (0, pad))
    NP = -(-N // (NS * L)) * (NS * L)

    norm = _make_norm_kernel(E_pad, NP)(row_p, col_p, w_p)
    xTp = jnp.pad(x.T, ((0, 0), (0, NP - N)))
    t1T, t2T = _make_prop_kernel(E_pad, NP, F_IN, C=E_pad // 32)(
        xTp, row_p, col_p, norm)
    return _matmul_relu(xTp, t1T, t2T, W, b, N, NP, F_IN, F_OUT)


# speedup vs baseline: 6.8762x; 2.1137x over previous
import functools

import jax
import jax.numpy as jnp
from jax import lax
from jax.experimental import pallas as pl
from jax.experimental.pallas import tpu as pltpu
from jax.experimental.pallas import tpu_sc as plsc

NC = 2
NS = 16
L = 16
NW = NC * NS


def _rsqrt_newton(d):
    bits = lax.bitcast_convert_type(d, jnp.int32)
    y = lax.bitcast_convert_type(
        jnp.int32(0x5F3759DF) - lax.shift_right_logical(bits, 1), jnp.float32)
    for _ in range(3):
        y = y * (1.5 - 0.5 * d * y * y)
    return y


def _zero_1d(ref, n):
    def z(i, _):
        ref[pl.ds(i * L, L)] = jnp.zeros((L,), jnp.float32)
        return 0
    lax.fori_loop(0, n // L, z, 0, unroll=8)


def _make_norm_kernel(E_pad, NP):
    EPT = E_pad // NS
    EPW = E_pad // NW
    SL = NP // NS
    mesh = plsc.VectorSubcoreMesh(
        core_axis_name="c", subcore_axis_name="s",
        num_cores=NC, num_subcores=NS)

    @functools.partial(
        pl.kernel, mesh=mesh,
        compiler_params=pltpu.CompilerParams(needs_layout_passes=False),
        out_type=jax.ShapeDtypeStruct((E_pad,), jnp.float32),
        scratch_types=[
            pltpu.VMEM((NP,), jnp.float32),
            pltpu.VMEM((NP,), jnp.float32),
            pltpu.VMEM((EPT,), jnp.int32),
            pltpu.VMEM((EPT,), jnp.float32),
            pltpu.VMEM((EPW,), jnp.int32),
            pltpu.VMEM((EPW,), jnp.float32),
            pltpu.VMEM((SL,), jnp.float32),
            pltpu.VMEM((SL,), jnp.float32),
            pltpu.VMEM_SHARED((NS, NP), jnp.float32),
            pltpu.VMEM_SHARED((NP,), jnp.float32),
        ],
    )
    def norm_kernel(row_hbm, col_hbm, w_hbm, norm_hbm,
                    deg_l, dinv_l, row_b, w_b, col_b, norm_b, tmp_b, acc_b,
                    deg_sh, dinv_sh):
        c = lax.axis_index("c")
        s = lax.axis_index("s")
        wid = s * NC + c

        _zero_1d(deg_l, NP)
        pltpu.sync_copy(row_hbm.at[pl.ds(s * EPT, EPT)], row_b)
        pltpu.sync_copy(w_hbm.at[pl.ds(s * EPT, EPT)], w_b)

        def acc_deg(g, _):
            r = row_b[pl.ds(g * L, L)]
            w = w_b[pl.ds(g * L, L)]
            plsc.addupdate_scatter(deg_l, [r], w)
            return 0
        lax.fori_loop(0, EPT // L, acc_deg, 0)
        pltpu.sync_copy(deg_l, deg_sh.at[s])
        plsc.subcore_barrier()

        base = s * SL
        _zero_1d(acc_b, SL)

        def red(j, _):
            pltpu.sync_copy(deg_sh.at[j, pl.ds(base, SL)], tmp_b)

            def addg(g, _):
                sl = pl.ds(g * L, L)
                acc_b[sl] = acc_b[sl] + tmp_b[sl]
                return 0
            lax.fori_loop(0, SL // L, addg, 0)
            return 0
        lax.fori_loop(0, NS, red, 0)

        def din(g, _):
            sl = pl.ds(g * L, L)
            d = acc_b[sl]
            acc_b[sl] = jnp.where(d > 0.0, _rsqrt_newton(d), 0.0)
            return 0
        lax.fori_loop(0, SL // L, din, 0)
        pltpu.sync_copy(acc_b, dinv_sh.at[pl.ds(base, SL)])
        plsc.subcore_barrier()

        pltpu.sync_copy(dinv_sh, dinv_l)
        ebase = wid * EPW
        pltpu.sync_copy(row_hbm.at[pl.ds(ebase, EPW)], row_b.at[pl.ds(0, EPW)])
        pltpu.sync_copy(col_hbm.at[pl.ds(ebase, EPW)], col_b)
        pltpu.sync_copy(w_hbm.at[pl.ds(ebase, EPW)], w_b.at[pl.ds(0, EPW)])

        def nrm(g, _):
            sl = pl.ds(g * L, L)
            dr = plsc.load_gather(dinv_l, [row_b[sl]])
            dc = plsc.load_gather(dinv_l, [col_b[sl]])
            norm_b[sl] = (-w_b[sl]) * dr * dc
            return 0
        lax.fori_loop(0, EPW // L, nrm, 0)
        pltpu.sync_copy(norm_b, norm_hbm.at[pl.ds(ebase, EPW)])

    return norm_kernel


def _make_prop_kernel(E_pad, NP, F, C):
    FPW = F // NW
    FG = 4
    assert FPW % FG == 0
    NCH = E_pad // C
    assert NCH % 2 == 0
    mesh = plsc.VectorSubcoreMesh(
        core_axis_name="c", subcore_axis_name="s",
        num_cores=NC, num_subcores=NS)

    @functools.partial(
        pl.kernel, mesh=mesh,
        compiler_params=pltpu.CompilerParams(needs_layout_passes=False),
        out_type=(jax.ShapeDtypeStruct((F, NP), jnp.float32),
                  jax.ShapeDtypeStruct((F, NP), jnp.float32)),
        scratch_types=(
            [pltpu.VMEM((NP,), jnp.float32) for _ in range(2 * FG)] +
            [pltpu.VMEM((C,), jnp.int32) for _ in range(2)] +
            [pltpu.VMEM((C,), jnp.int32) for _ in range(2)] +
            [pltpu.VMEM((C,), jnp.float32) for _ in range(2)] +
            [pltpu.SemaphoreType.DMA for _ in range(6)]
        ),
    )
    def prop_kernel(xT, row_hbm, col_hbm, norm_hbm, t1T, t2T,
                    a0, a1, a2, a3, b0, b1, b2, b3,
                    rb0, rb1, cb0, cb1, nb0, nb1,
                    sr0, sr1, sc0, sc1, sn0, sn1):
        A = [a0, a1, a2, a3]
        B = [b0, b1, b2, b3]
        rbs, cbs, nbs = [rb0, rb1], [cb0, cb1], [nb0, nb1]
        srs, scs, sns = [sr0, sr1], [sc0, sc1], [sn0, sn1]
        c = lax.axis_index("c")
        s = lax.axis_index("s")
        wid = s * NC + c
        f0 = wid * FPW

        def start(ci, slot):
            eb = ci * C
            pltpu.async_copy(row_hbm.at[pl.ds(eb, C)], rbs[slot], srs[slot])
            pltpu.async_copy(col_hbm.at[pl.ds(eb, C)], cbs[slot], scs[slot])
            pltpu.async_copy(norm_hbm.at[pl.ds(eb, C)], nbs[slot], sns[slot])

        def wait(slot):
            pltpu.make_async_copy(
                row_hbm.at[pl.ds(0, C)], rbs[slot], srs[slot]).wait()
            pltpu.make_async_copy(
                col_hbm.at[pl.ds(0, C)], cbs[slot], scs[slot]).wait()
            pltpu.make_async_copy(
                norm_hbm.at[pl.ds(0, C)], nbs[slot], sns[slot]).wait()

        def edge_sweep(src, dst):
            def process(slot):
                rb, cb, nb = rbs[slot], cbs[slot], nbs[slot]

                @plsc.parallel_loop(0, C // L, unroll=8)
                def grp(g):
                    sl = pl.ds(g * L, L)
                    cc = cb[sl]
                    r = rb[sl]
                    nv = nb[sl]
                    vs = [plsc.load_gather(src[ff], [cc]) * nv
                          for ff in range(FG)]
                    for ff in range(FG):
                        plsc.addupdate_scatter(dst[ff], [r], vs[ff])

            start(0, 0)

            def body2(ci2, _):
                ci = ci2 * 2
                start(ci + 1, 1)
                wait(0)
                process(0)

                @pl.when(ci2 < NCH // 2 - 1)
                def _():
                    start(ci + 2, 0)
                wait(1)
                process(1)
                return 0
            lax.fori_loop(0, NCH // 2, body2, 0)

        for fg in range(FPW // FG):
            fbase = f0 + fg * FG
            for ff in range(FG):
                pltpu.sync_copy(xT.at[fbase + ff], A[ff])
                _zero_1d(B[ff], NP)
            edge_sweep(A, B)
            for ff in range(FG):
                pltpu.sync_copy(B[ff], t1T.at[fbase + ff])
            for ff in range(FG):
                _zero_1d(A[ff], NP)
            edge_sweep(B, A)
            for ff in range(FG):
                pltpu.sync_copy(xT.at[fbase + ff], B[ff])
                a, xt = A[ff], B[ff]

                def comb(g, _):
                    sl = pl.ds(g * L, L)
                    a[sl] = 2.0 * a[sl] - xt[sl]
                    return 0
                lax.fori_loop(0, NP // L, comb, 0, unroll=8)
                pltpu.sync_copy(a, t2T.at[fbase + ff])

    return prop_kernel


def _matmul_relu(xT, t1T, t2T, W, b, N, NP, F_IN, F_OUT, BN=1024):
    def mm_kernel(x_ref, t1_ref, t2_ref, w0, w1, w2, b_ref, o_ref):
        dn = (((0,), (0,)), ((), ()))
        acc = lax.dot_general(x_ref[...], w0[...], dn,
                              preferred_element_type=jnp.float32)
        acc = acc + lax.dot_general(t1_ref[...], w1[...], dn,
                                    preferred_element_type=jnp.float32)
        acc = acc + lax.dot_general(t2_ref[...], w2[...], dn,
                                    preferred_element_type=jnp.float32)
        acc = acc + b_ref[...]
        o_ref[...] = jnp.maximum(acc, 0.0)

    grid = (pl.cdiv(N, BN),)
    fspec = pl.BlockSpec((F_IN, BN), lambda i: (0, i))
    wspec = pl.BlockSpec((F_IN, F_OUT), lambda i: (0, 0))
    return pl.pallas_call(
        mm_kernel,
        grid=grid,
        in_specs=[fspec, fspec, fspec, wspec, wspec, wspec,
                  pl.BlockSpec((1, F_OUT), lambda i: (0, 0))],
        out_specs=pl.BlockSpec((BN, F_OUT), lambda i: (i, 0)),
        out_shape=jax.ShapeDtypeStruct((N, F_OUT), jnp.float32),
    )(xT, t1T, t2T, W[0], W[1], W[2], b.reshape(1, F_OUT))


def kernel(x, edge_index, edge_weight, W, b):
    N, F_IN = x.shape
    F_OUT = W.shape[2]
    E = edge_weight.shape[0]

    E_pad = -(-E // (NW * 128)) * (NW * 128)
    pad = E_pad - E
    row_p = jnp.pad(edge_index[0], (0, pad))
    col_p = jnp.pad(edge_index[1], (0, pad))
    w_p = jnp.pad(edge_weight, (0, pad))
    NP = -(-N // (NS * L)) * (NS * L)

    norm = _make_norm_kernel(E_pad, NP)(row_p, col_p, w_p)
    xTp = jnp.pad(x.T, ((0, 0), (0, NP - N)))
    t1T, t2T = _make_prop_kernel(E_pad, NP, F_IN, C=E_pad // 32)(
        xTp, row_p, col_p, norm)
    return _matmul_relu(xTp, t1T, t2T, W, b, N, NP, F_IN, F_OUT)


# speedup vs baseline: 7.4787x; 1.0876x over previous
import functools

import jax
import jax.numpy as jnp
from jax import lax
from jax.experimental import pallas as pl
from jax.experimental.pallas import tpu as pltpu
from jax.experimental.pallas import tpu_sc as plsc

NC = 2
NS = 16
L = 16
NW = NC * NS


def _rsqrt_newton(d):
    bits = lax.bitcast_convert_type(d, jnp.int32)
    y = lax.bitcast_convert_type(
        jnp.int32(0x5F3759DF) - lax.shift_right_logical(bits, 1), jnp.float32)
    for _ in range(3):
        y = y * (1.5 - 0.5 * d * y * y)
    return y


def _zero_1d(ref, n):
    @plsc.parallel_loop(0, n // L, unroll=8)
    def z(i):
        ref[pl.ds(i * L, L)] = jnp.zeros((L,), jnp.float32)


def _make_norm_kernel(E_pad, NP, SH):
    EPT = E_pad // NS
    EPW = E_pad // NW
    SL = NP // NS
    MASK = (1 << SH) - 1
    mesh = plsc.VectorSubcoreMesh(
        core_axis_name="c", subcore_axis_name="s",
        num_cores=NC, num_subcores=NS)

    @functools.partial(
        pl.kernel, mesh=mesh,
        compiler_params=pltpu.CompilerParams(needs_layout_passes=False),
        out_type=jax.ShapeDtypeStruct((E_pad,), jnp.float32),
        scratch_types=[
            pltpu.VMEM((NP,), jnp.float32),
            pltpu.VMEM((NP,), jnp.float32),
            pltpu.VMEM((EPT,), jnp.int32),
            pltpu.VMEM((EPT,), jnp.float32),
            pltpu.VMEM((EPW,), jnp.float32),
            pltpu.VMEM((SL,), jnp.float32),
            pltpu.VMEM((SL,), jnp.float32),
            pltpu.VMEM_SHARED((NS, NP), jnp.float32),
            pltpu.VMEM_SHARED((NP,), jnp.float32),
        ],
    )
    def norm_kernel(rc_hbm, w_hbm, norm_hbm,
                    deg_l, dinv_l, rc_b, w_b, norm_b, tmp_b, acc_b,
                    deg_sh, dinv_sh):
        c = lax.axis_index("c")
        s = lax.axis_index("s")
        wid = s * NC + c

        _zero_1d(deg_l, NP)
        pltpu.sync_copy(rc_hbm.at[pl.ds(s * EPT, EPT)], rc_b)
        pltpu.sync_copy(w_hbm.at[pl.ds(s * EPT, EPT)], w_b)

        @plsc.parallel_loop(0, EPT // L, unroll=8)
        def acc_deg(g):
            sl = pl.ds(g * L, L)
            r = lax.shift_right_logical(rc_b[sl], SH)
            plsc.addupdate_scatter(deg_l, [r], w_b[sl])

        pltpu.sync_copy(deg_l, deg_sh.at[s])
        plsc.subcore_barrier()

        base = s * SL
        _zero_1d(acc_b, SL)

        def red(j, _):
            pltpu.sync_copy(deg_sh.at[j, pl.ds(base, SL)], tmp_b)

            @plsc.parallel_loop(0, SL // L, unroll=8)
            def addg(g):
                sl = pl.ds(g * L, L)
                acc_b[sl] = acc_b[sl] + tmp_b[sl]
            return 0
        lax.fori_loop(0, NS, red, 0)

        @plsc.parallel_loop(0, SL // L, unroll=4)
        def din(g):
            sl = pl.ds(g * L, L)
            d = acc_b[sl]
            acc_b[sl] = jnp.where(d > 0.0, _rsqrt_newton(d), 0.0)

        pltpu.sync_copy(acc_b, dinv_sh.at[pl.ds(base, SL)])
        plsc.subcore_barrier()

        pltpu.sync_copy(dinv_sh, dinv_l)
        ebase = wid * EPW
        pltpu.sync_copy(rc_hbm.at[pl.ds(ebase, EPW)], rc_b.at[pl.ds(0, EPW)])
        pltpu.sync_copy(w_hbm.at[pl.ds(ebase, EPW)], w_b.at[pl.ds(0, EPW)])

        @plsc.parallel_loop(0, EPW // L, unroll=8)
        def nrm(g):
            sl = pl.ds(g * L, L)
            rc = rc_b[sl]
            dr = plsc.load_gather(dinv_l, [lax.shift_right_logical(rc, SH)])
            dc = plsc.load_gather(dinv_l, [rc & MASK])
            norm_b[sl] = (-w_b[sl]) * dr * dc

        pltpu.sync_copy(norm_b, norm_hbm.at[pl.ds(ebase, EPW)])

    return norm_kernel


def _make_prop_kernel(E_pad, NP, F, C, SH):
    FPW = F // NW
    FG = 4
    assert FPW % FG == 0
    NCH = E_pad // C
    assert NCH % 2 == 0
    MASK = (1 << SH) - 1
    mesh = plsc.VectorSubcoreMesh(
        core_axis_name="c", subcore_axis_name="s",
        num_cores=NC, num_subcores=NS)

    @functools.partial(
        pl.kernel, mesh=mesh,
        compiler_params=pltpu.CompilerParams(needs_layout_passes=False),
        out_type=(jax.ShapeDtypeStruct((F, NP), jnp.float32),
                  jax.ShapeDtypeStruct((F, NP), jnp.float32)),
        scratch_types=(
            [pltpu.VMEM((NP,), jnp.float32) for _ in range(2 * FG)] +
            [pltpu.VMEM((C,), jnp.int32) for _ in range(2)] +
            [pltpu.VMEM((C,), jnp.float32) for _ in range(2)] +
            [pltpu.SemaphoreType.DMA for _ in range(4)]
        ),
    )
    def prop_kernel(xT, rc_hbm, norm_hbm, t1T, t2T,
                    a0, a1, a2, a3, b0, b1, b2, b3,
                    rc0, rc1, nb0, nb1,
                    sr0, sr1, sn0, sn1):
        A = [a0, a1, a2, a3]
        B = [b0, b1, b2, b3]
        rcs, nbs = [rc0, rc1], [nb0, nb1]
        srs, sns = [sr0, sr1], [sn0, sn1]
        c = lax.axis_index("c")
        s = lax.axis_index("s")
        wid = s * NC + c
        f0 = wid * FPW

        def start(ci, slot):
            eb = ci * C
            pltpu.async_copy(rc_hbm.at[pl.ds(eb, C)], rcs[slot], srs[slot])
            pltpu.async_copy(norm_hbm.at[pl.ds(eb, C)], nbs[slot], sns[slot])

        def wait(slot):
            pltpu.make_async_copy(
                rc_hbm.at[pl.ds(0, C)], rcs[slot], srs[slot]).wait()
            pltpu.make_async_copy(
                norm_hbm.at[pl.ds(0, C)], nbs[slot], sns[slot]).wait()

        def edge_sweep(src, dst):
            def process(slot):
                rcb, nb = rcs[slot], nbs[slot]

                @plsc.parallel_loop(0, C // L, unroll=8)
                def grp(g):
                    sl = pl.ds(g * L, L)
                    rc = rcb[sl]
                    nv = nb[sl]
                    cc = rc & MASK
                    r = lax.shift_right_logical(rc, SH)
                    vs = [plsc.load_gather(src[ff], [cc]) * nv
                          for ff in range(FG)]
                    for ff in range(FG):
                        plsc.addupdate_scatter(dst[ff], [r], vs[ff])

            start(0, 0)

            def body2(ci2, _):
                ci = ci2 * 2
                start(ci + 1, 1)
                wait(0)
                process(0)

                @pl.when(ci2 < NCH // 2 - 1)
                def _():
                    start(ci + 2, 0)
                wait(1)
                process(1)
                return 0
            lax.fori_loop(0, NCH // 2, body2, 0)

        for fg in range(FPW // FG):
            fbase = f0 + fg * FG
            for ff in range(FG):
                pltpu.sync_copy(xT.at[fbase + ff], A[ff])
                _zero_1d(B[ff], NP)
            edge_sweep(A, B)
            for ff in range(FG):
                pltpu.sync_copy(B[ff], t1T.at[fbase + ff])
            for ff in range(FG):
                _zero_1d(A[ff], NP)
            edge_sweep(B, A)
            for ff in range(FG):
                pltpu.sync_copy(xT.at[fbase + ff], B[ff])
                a, xt = A[ff], B[ff]

                @plsc.parallel_loop(0, NP // L, unroll=8)
                def comb(g):
                    sl = pl.ds(g * L, L)
                    a[sl] = 2.0 * a[sl] - xt[sl]

                pltpu.sync_copy(a, t2T.at[fbase + ff])

    return prop_kernel


def _matmul_relu(xT, t1T, t2T, W, b, N, NP, F_IN, F_OUT, BN=1024):
    def mm_kernel(x_ref, t1_ref, t2_ref, w0, w1, w2, b_ref, o_ref):
        dn = (((0,), (0,)), ((), ()))
        acc = lax.dot_general(x_ref[...], w0[...], dn,
                              preferred_element_type=jnp.float32)
        acc = acc + lax.dot_general(t1_ref[...], w1[...], dn,
                                    preferred_element_type=jnp.float32)
        acc = acc + lax.dot_general(t2_ref[...], w2[...], dn,
                                    preferred_element_type=jnp.float32)
        acc = acc + b_ref[...]
        o_ref[...] = jnp.maximum(acc, 0.0)

    grid = (pl.cdiv(N, BN),)
    fspec = pl.BlockSpec((F_IN, BN), lambda i: (0, i))
    wspec = pl.BlockSpec((F_IN, F_OUT), lambda i: (0, 0))
    return pl.pallas_call(
        mm_kernel,
        grid=grid,
        in_specs=[fspec, fspec, fspec, wspec, wspec, wspec,
                  pl.BlockSpec((1, F_OUT), lambda i: (0, 0))],
        out_specs=pl.BlockSpec((BN, F_OUT), lambda i: (i, 0)),
        out_shape=jax.ShapeDtypeStruct((N, F_OUT), jnp.float32),
    )(xT, t1T, t2T, W[0], W[1], W[2], b.reshape(1, F_OUT))


def kernel(x, edge_index, edge_weight, W, b):
    N, F_IN = x.shape
    F_OUT = W.shape[2]
    E = edge_weight.shape[0]

    E_pad = -(-E // (NW * 128)) * (NW * 128)
    pad = E_pad - E
    SH = max((N - 1).bit_length(), 1)
    assert 2 * SH <= 31
    rc = jnp.pad(edge_index[0] << SH | edge_index[1], (0, pad))
    w_p = jnp.pad(edge_weight, (0, pad))
    NP = -(-N // (NS * L)) * (NS * L)

    norm = _make_norm_kernel(E_pad, NP, SH)(rc, w_p)
    xTp = jnp.pad(x.T, ((0, 0), (0, NP - N)))
    t1T, t2T = _make_prop_kernel(E_pad, NP, F_IN, E_pad // 32, SH)(
        xTp, rc, norm)
    return _matmul_relu(xTp, t1T, t2T, W, b, N, NP, F_IN, F_OUT)
